# Initial kernel scaffold; baseline (speedup 1.0000x reference)
import functools

import jax
import jax.numpy as jnp
from jax import lax
from jax.experimental import pallas as pl
from jax.experimental.pallas import tpu as pltpu
from jax.experimental.pallas import tpu_sc as plsc

N_NODES = 10000
N_EDGES = 320000
D = 128
D_OUT = 64

NP = 10240
EP = 327680
PAD_NODE = 10200

NW = 32
EDGES_PER_TILE = EP // NW
K = 128
CHUNKS = EDGES_PER_TILE // K
ROWS_PER_TILE = NP // 16
ACC_W = 144

_R = 2048
_G = NP // _R
_AR = _R // D


def _tc_first_body(x_ref, ws_ref, wd_ref, as_ref, ad_ref, hs_ref, als_ref, ald_ref):
    x = x_ref[...]
    hs = jnp.dot(x, ws_ref[...], preferred_element_type=jnp.float32)
    hs_ref[...] = hs
    als = jnp.dot(hs, as_ref[...], preferred_element_type=jnp.float32)
    als_ref[...] = als.reshape(_AR, D)
    v = jnp.dot(wd_ref[...], ad_ref[...], preferred_element_type=jnp.float32)
    ald_ref[...] = jnp.dot(x, v, preferred_element_type=jnp.float32).reshape(_AR, D)


def _tc_mid_body(acc_ref, b_ref, w_ref, as_ref, ad_ref, hs_ref, als_ref, ald_ref):
    num = acc_ref[0, :, :D] + acc_ref[1, :, :D]
    s = acc_ref[0, :, D:D + 1] + acc_ref[1, :, D:D + 1]
    h = jnp.maximum(jnp.where(s > 0.0, num / s, 0.0) + b_ref[...], 0.0)
    hs = jnp.dot(h, w_ref[...], preferred_element_type=jnp.float32)
    hs_ref[...] = hs
    als = jnp.dot(hs, as_ref[...], preferred_element_type=jnp.float32)
    als_ref[...] = als.reshape(_AR, D)
    v = jnp.dot(w_ref[...], ad_ref[...], preferred_element_type=jnp.float32)
    ald_ref[...] = jnp.dot(h, v, preferred_element_type=jnp.float32).reshape(_AR, D)


def _tc_last_body(acc_ref, b_ref, w1_ref, b1_ref, w2_ref, b2_ref, out_ref):
    num = acc_ref[0, :, :D] + acc_ref[1, :, :D]
    s = acc_ref[0, :, D:D + 1] + acc_ref[1, :, D:D + 1]
    h = jnp.maximum(jnp.where(s > 0.0, num / s, 0.0) + b_ref[...], 0.0)
    h = jnp.maximum(jnp.dot(h, w1_ref[...], preferred_element_type=jnp.float32)
                    + b1_ref[...], 0.0)
    out_ref[...] = jnp.dot(h, w2_ref[...], preferred_element_type=jnp.float32) + b2_ref[...]


def _row_blk(i):
    return (i, 0)


def _acc_blk(i):
    return (0, i, 0)


def _full_blk(i):
    return (0, 0)


_W_SPEC = pl.BlockSpec((D, D), _full_blk)
_A_SPEC = pl.BlockSpec((D, 1), _full_blk)
_B_SPEC = pl.BlockSpec((1, D), _full_blk)
_H_SPEC = pl.BlockSpec((_R, D), _row_blk)
_AL_SPEC = pl.BlockSpec((_AR, D), _row_blk)
_ACC_SPEC = pl.BlockSpec((2, _R, ACC_W), _acc_blk)

_PROJ_OUT = (jax.ShapeDtypeStruct((NP, D), jnp.float32),
             jax.ShapeDtypeStruct((NP // D, D), jnp.float32),
             jax.ShapeDtypeStruct((NP // D, D), jnp.float32))

_tc_first = pl.pallas_call(
    _tc_first_body, grid=(_G,),
    in_specs=[_H_SPEC, _W_SPEC, _W_SPEC, _A_SPEC, _A_SPEC],
    out_specs=[_H_SPEC, _AL_SPEC, _AL_SPEC],
    out_shape=_PROJ_OUT)

_tc_mid = pl.pallas_call(
    _tc_mid_body, grid=(_G,),
    in_specs=[_ACC_SPEC, _B_SPEC, _W_SPEC, _A_SPEC, _A_SPEC],
    out_specs=[_H_SPEC, _AL_SPEC, _AL_SPEC],
    out_shape=_PROJ_OUT)

_tc_last = pl.pallas_call(
    _tc_last_body, grid=(_G,),
    in_specs=[_ACC_SPEC, _B_SPEC, _W_SPEC, _B_SPEC, _W_SPEC, _B_SPEC],
    out_specs=_H_SPEC,
    out_shape=jax.ShapeDtypeStruct((NP, D), jnp.float32))


def _sc_edge_body(hs_hbm, src_hbm, dst_hbm, as_hbm, ad_hbm, out_hbm,
                  src_c, dst_c, as_v, ad_v, ee_v, rows_v, sc_v, acc_sp, sem):
    cid = lax.axis_index("c")
    sid = lax.axis_index("s")
    wid = sid * 2 + cid

    pltpu.sync_copy(src_hbm.at[pl.ds(wid * CHUNKS, CHUNKS)], src_c)
    pltpu.sync_copy(dst_hbm.at[pl.ds(wid * CHUNKS, CHUNKS)], dst_c)
    pltpu.sync_copy(as_hbm, as_v)
    pltpu.sync_copy(ad_hbm, ad_v)

    zero16 = jnp.zeros((16,), jnp.float32)

    def _z(r, carry):
        for c in range(ACC_W // 16):
            sc_v[r, pl.ds(c * 16, 16)] = zero16
        return carry
    lax.fori_loop(0, K, _z, 0)

    for t in range(ROWS_PER_TILE // K):
        pltpu.sync_copy(sc_v, acc_sp.at[pl.ds(sid * ROWS_PER_TILE + t * K, K)])
    plsc.subcore_barrier()

    lane0 = lax.broadcasted_iota(jnp.int32, (16,), 0) == 0

    def _chunk(j, carry):
        def _ee(t, c2):
            sv = src_c[j, pl.ds(t * 16, 16)]
            dv = dst_c[j, pl.ds(t * 16, 16)]
            e = plsc.load_gather(as_v, [sv]) + plsc.load_gather(ad_v, [dv])
            e = jnp.where(e > 0.0, e, 0.2 * e)
            ee_v[pl.ds(t * 16, 16)] = jnp.exp(e)
            return c2
        lax.fori_loop(0, K // 16, _ee, 0)

        pltpu.async_copy(hs_hbm.at[src_c.at[j]], rows_v, sem).wait()

        def _row(i, c2):
            w = ee_v[i]
            for c in range(D // 16):
                sc_v[i, pl.ds(c * 16, 16)] = rows_v[i, pl.ds(c * 16, 16)] * w
            sc_v[i, pl.ds(D, 16)] = jnp.where(lane0, w, 0.0)
            return c2
        lax.fori_loop(0, K, _row, 0)

        pltpu.sync_copy(sc_v, acc_sp.at[dst_c.at[j]], add=True)
        return carry
    lax.fori_loop(0, CHUNKS, _chunk, 0)

    plsc.subcore_barrier()
    pltpu.sync_copy(acc_sp.at[pl.ds(sid * ROWS_PER_TILE, ROWS_PER_TILE)],
                    out_hbm.at[cid, pl.ds(sid * ROWS_PER_TILE, ROWS_PER_TILE)])


_sc_edge = functools.partial(
    pl.kernel,
    out_type=jax.ShapeDtypeStruct((2, NP, ACC_W), jnp.float32),
    mesh=plsc.VectorSubcoreMesh(core_axis_name="c", subcore_axis_name="s"),
    scratch_types=[
        pltpu.VMEM((CHUNKS, K), jnp.int32),
        pltpu.VMEM((CHUNKS, K), jnp.int32),
        pltpu.VMEM((NP,), jnp.float32),
        pltpu.VMEM((NP,), jnp.float32),
        pltpu.VMEM((K,), jnp.float32),
        pltpu.VMEM((K, D), jnp.float32),
        pltpu.VMEM((K, ACC_W), jnp.float32),
        pltpu.VMEM_SHARED((NP, ACC_W), jnp.float32),
        pltpu.SemaphoreType.DMA,
    ])(_sc_edge_body)


def kernel(x, edge_index, W1s, W1d, a1s, a1d, b1, W2, a2s, a2d, b2,
           W3, a3s, a3d, b3, lin1_W, lin1_b, lin2_W, lin2_b):
    f32 = jnp.float32
    x_p = jnp.zeros((NP, D), f32).at[:N_NODES].set(x)
    pad = jnp.full((EP - N_EDGES,), PAD_NODE, jnp.int32)
    src = jnp.concatenate([edge_index[0], pad]).reshape(EP // K, K)
    dst = jnp.concatenate([edge_index[1], pad]).reshape(EP // K, K)

    def col(a):
        return a.reshape(D, 1)

    def row(a, w=D):
        return a.reshape(1, w)

    hs, als, ald = _tc_first(x_p, W1s, W1d, col(a1s), col(a1d))
    acc = _sc_edge(hs, src, dst, als.reshape(NP), ald.reshape(NP))
    hs, als, ald = _tc_mid(acc, row(b1), W2, col(a2s), col(a2d))
    acc = _sc_edge(hs, src, dst, als.reshape(NP), ald.reshape(NP))
    hs, als, ald = _tc_mid(acc, row(b2), W3, col(a3s), col(a3d))
    acc = _sc_edge(hs, src, dst, als.reshape(NP), ald.reshape(NP))

    w2p = jnp.zeros((D, D), f32).at[:, :D_OUT].set(lin2_W)
    b2p = jnp.zeros((D,), f32).at[:D_OUT].set(lin2_b)
    out = _tc_last(acc, row(b3), lin1_W, row(lin1_b), w2p, row(b2p))
    return out[:N_NODES, :D_OUT]


# speedup vs baseline: 15.1807x; 15.1807x over previous
import functools

import jax
import jax.numpy as jnp
from jax import lax
from jax.experimental import pallas as pl
from jax.experimental.pallas import tpu as pltpu
from jax.experimental.pallas import tpu_sc as plsc

N_NODES = 10000
N_EDGES = 320000
D = 128
D_OUT = 64

NP = 10240
EP = 327680
PAD_NODE = 10200

NW = 32
EDGES_PER_TILE = EP // NW
K = 64
SB = 8
CHUNKS = EDGES_PER_TILE // K
ROWS_PER_TILE = NP // 16
ZR = 128

_R = 2048
_G = NP // _R
_AR = _R // D


def _tc_first_body(x_ref, ws_ref, wd_ref, as_ref, ad_ref, hs_ref, als_ref, ald_ref):
    x = x_ref[...]
    hs = jnp.dot(x, ws_ref[...], preferred_element_type=jnp.float32)
    hs_ref[...] = hs
    als = jnp.dot(hs, as_ref[...], preferred_element_type=jnp.float32)
    als_ref[...] = als.reshape(_AR, D)
    v = jnp.dot(wd_ref[...], ad_ref[...], preferred_element_type=jnp.float32)
    ald_ref[...] = jnp.dot(x, v, preferred_element_type=jnp.float32).reshape(_AR, D)


def _tc_mid_body(acc_ref, s_ref, b_ref, w_ref, as_ref, ad_ref, hs_ref, als_ref, ald_ref):
    num = acc_ref[0] + acc_ref[1]
    s = jnp.sum(s_ref[...], axis=1, keepdims=True)
    h = jnp.maximum(jnp.where(s > 0.0, num / s, 0.0) + b_ref[...], 0.0)
    hs = jnp.dot(h, w_ref[...], preferred_element_type=jnp.float32)
    hs_ref[...] = hs
    als = jnp.dot(hs, as_ref[...], preferred_element_type=jnp.float32)
    als_ref[...] = als.reshape(_AR, D)
    v = jnp.dot(w_ref[...], ad_ref[...], preferred_element_type=jnp.float32)
    ald_ref[...] = jnp.dot(h, v, preferred_element_type=jnp.float32).reshape(_AR, D)


def _tc_last_body(acc_ref, s_ref, b_ref, w1_ref, b1_ref, w2_ref, b2_ref, out_ref):
    num = acc_ref[0] + acc_ref[1]
    s = jnp.sum(s_ref[...], axis=1, keepdims=True)
    h = jnp.maximum(jnp.where(s > 0.0, num / s, 0.0) + b_ref[...], 0.0)
    h = jnp.maximum(jnp.dot(h, w1_ref[...], preferred_element_type=jnp.float32)
                    + b1_ref[...], 0.0)
    out_ref[...] = jnp.dot(h, w2_ref[...], preferred_element_type=jnp.float32) + b2_ref[...]


def _row_blk(i):
    return (i, 0)


def _acc_blk(i):
    return (0, i, 0)


def _full_blk(i):
    return (0, 0)


_W_SPEC = pl.BlockSpec((D, D), _full_blk)
_A_SPEC = pl.BlockSpec((D, 1), _full_blk)
_B_SPEC = pl.BlockSpec((1, D), _full_blk)
_H_SPEC = pl.BlockSpec((_R, D), _row_blk)
_AL_SPEC = pl.BlockSpec((_AR, D), _row_blk)
_ACC_SPEC = pl.BlockSpec((2, _R, D), _acc_blk)
_S_SPEC = pl.BlockSpec((_R, NW), _row_blk)

_PROJ_OUT = (jax.ShapeDtypeStruct((NP, D), jnp.float32),
             jax.ShapeDtypeStruct((NP // D, D), jnp.float32),
             jax.ShapeDtypeStruct((NP // D, D), jnp.float32))

_tc_first = pl.pallas_call(
    _tc_first_body, grid=(_G,),
    in_specs=[_H_SPEC, _W_SPEC, _W_SPEC, _A_SPEC, _A_SPEC],
    out_specs=[_H_SPEC, _AL_SPEC, _AL_SPEC],
    out_shape=_PROJ_OUT)

_tc_mid = pl.pallas_call(
    _tc_mid_body, grid=(_G,),
    in_specs=[_ACC_SPEC, _S_SPEC, _B_SPEC, _W_SPEC, _A_SPEC, _A_SPEC],
    out_specs=[_H_SPEC, _AL_SPEC, _AL_SPEC],
    out_shape=_PROJ_OUT)

_tc_last = pl.pallas_call(
    _tc_last_body, grid=(_G,),
    in_specs=[_ACC_SPEC, _S_SPEC, _B_SPEC, _W_SPEC, _B_SPEC, _W_SPEC, _B_SPEC],
    out_specs=_H_SPEC,
    out_shape=jax.ShapeDtypeStruct((NP, D), jnp.float32))


def _sc_edge_body(hs_hbm, src_hbm, dst_hbm, as_hbm, ad_hbm, out_hbm, s_hbm,
                  src_sb, dst_sb, as_v, ad_v, ee_v, rows_v, s_loc, acc_sp, sem):
    cid = lax.axis_index("c")
    sid = lax.axis_index("s")
    wid = sid * 2 + cid

    pltpu.sync_copy(as_hbm, as_v)
    pltpu.sync_copy(ad_hbm, ad_v)

    zero16 = jnp.zeros((16,), jnp.float32)

    def _z(r, carry):
        for c in range(D // 16):
            rows_v[r, pl.ds(c * 16, 16)] = zero16
        return carry
    lax.fori_loop(0, K, _z, 0)

    def _zs(r, carry):
        for c in range(128 // 16):
            s_loc[r, pl.ds(c * 16, 16)] = zero16
        return carry
    lax.fori_loop(0, NP // 128, _zs, 0)

    def _za(t, carry):
        pltpu.sync_copy(rows_v, acc_sp.at[pl.ds(sid * ROWS_PER_TILE + t * K, K)])
        return carry
    lax.fori_loop(0, ROWS_PER_TILE // K, _za, 0)
    plsc.subcore_barrier()

    def _sblock(sb, carry):
        base = wid * CHUNKS + sb * SB
        pltpu.sync_copy(src_hbm.at[pl.ds(base, SB)], src_sb)
        pltpu.sync_copy(dst_hbm.at[pl.ds(base, SB)], dst_sb)

        def _chunk(j, c1):
            def _ee(t, c2):
                sv = src_sb[j, pl.ds(t * 16, 16)]
                dv = dst_sb[j, pl.ds(t * 16, 16)]
                e = (plsc.load_gather(as_v, [sv >> 7, sv & 127])
                     + plsc.load_gather(ad_v, [dv >> 7, dv & 127]))
                e = jnp.where(e > 0.0, e, 0.2 * e)
                ee = jnp.exp(e)
                ee_v[t] = ee
                plsc.addupdate_scatter(s_loc, [dv >> 7, dv & 127], ee)
                return c2
            lax.fori_loop(0, K // 16, _ee, 0)

            pltpu.async_copy(hs_hbm.at[src_sb.at[j]], rows_v, sem).wait()

            def _row(i, c2):
                w = plsc.load_gather(
                    ee_v, [jnp.broadcast_to(i >> 4, (16,)),
                           jnp.broadcast_to(i & 15, (16,))])
                for c in range(D // 16):
                    rows_v[i, pl.ds(c * 16, 16)] = rows_v[i, pl.ds(c * 16, 16)] * w
                return c2
            lax.fori_loop(0, K, _row, 0)

            pltpu.sync_copy(rows_v, acc_sp.at[dst_sb.at[j]], add=True)
            return c1
        lax.fori_loop(0, SB, _chunk, 0)
        return carry
    lax.fori_loop(0, CHUNKS // SB, _sblock, 0)

    pltpu.sync_copy(s_loc, s_hbm.at[wid])
    plsc.subcore_barrier()

    def _wb(t, carry):
        pltpu.sync_copy(
            acc_sp.at[pl.ds(sid * ROWS_PER_TILE + t * ZR, ZR)],
            out_hbm.at[cid, pl.ds(sid * ROWS_PER_TILE + t * ZR, ZR)])
        return carry
    lax.fori_loop(0, ROWS_PER_TILE // ZR, _wb, 0)


_sc_edge = functools.partial(
    pl.kernel,
    out_type=(jax.ShapeDtypeStruct((2, NP, D), jnp.float32),
              jax.ShapeDtypeStruct((NW, NP // 128, 128), jnp.float32)),
    mesh=plsc.VectorSubcoreMesh(core_axis_name="c", subcore_axis_name="s"),
    scratch_types=[
        pltpu.VMEM((SB, K), jnp.int32),
        pltpu.VMEM((SB, K), jnp.int32),
        pltpu.VMEM((NP // 128, 128), jnp.float32),
        pltpu.VMEM((NP // 128, 128), jnp.float32),
        pltpu.VMEM((K // 16, 16), jnp.float32),
        pltpu.VMEM((K, D), jnp.float32),
        pltpu.VMEM((NP // 128, 128), jnp.float32),
        pltpu.VMEM_SHARED((NP, D), jnp.float32),
        pltpu.SemaphoreType.DMA,
    ],
    compiler_params=pltpu.CompilerParams(needs_layout_passes=False),
    )(_sc_edge_body)


def kernel(x, edge_index, W1s, W1d, a1s, a1d, b1, W2, a2s, a2d, b2,
           W3, a3s, a3d, b3, lin1_W, lin1_b, lin2_W, lin2_b):
    f32 = jnp.float32
    x_p = jnp.zeros((NP, D), f32).at[:N_NODES].set(x)
    pad = jnp.full((EP - N_EDGES,), PAD_NODE, jnp.int32)
    src = jnp.concatenate([edge_index[0], pad]).reshape(EP // K, K)
    dst = jnp.concatenate([edge_index[1], pad]).reshape(EP // K, K)

    def col(a):
        return a.reshape(D, 1)

    def row(a, w=D):
        return a.reshape(1, w)

    def tr(sv):
        return sv.reshape(NW, NP).transpose(1, 0)

    hs, als, ald = _tc_first(x_p, W1s, W1d, col(a1s), col(a1d))
    acc, sv = _sc_edge(hs, src, dst, als, ald)
    hs, als, ald = _tc_mid(acc, tr(sv), row(b1), W2, col(a2s), col(a2d))
    acc, sv = _sc_edge(hs, src, dst, als, ald)
    hs, als, ald = _tc_mid(acc, tr(sv), row(b2), W3, col(a3s), col(a3d))
    acc, sv = _sc_edge(hs, src, dst, als, ald)

    w2p = jnp.zeros((D, D), f32).at[:, :D_OUT].set(lin2_W)
    b2p = jnp.zeros((D,), f32).at[:D_OUT].set(lin2_b)
    out = _tc_last(acc, tr(sv), row(b3), lin1_W, row(lin1_b), w2p, row(b2p))
    return out[:N_NODES, :D_OUT]


# speedup vs baseline: 18.4196x; 1.2134x over previous
import functools

import jax
import jax.numpy as jnp
from jax import lax
from jax.experimental import pallas as pl
from jax.experimental.pallas import tpu as pltpu
from jax.experimental.pallas import tpu_sc as plsc

N_NODES = 10000
N_EDGES = 320000
D = 128
D_OUT = 64

NP = 10240
EP = 327680
PAD_NODE = 10100

NW = 32
EDGES_PER_TILE = EP // NW
K = 64
SB = 4
AL_R = 80
CHUNKS = EDGES_PER_TILE // K
ROWS_PER_TILE = NP // 16
ZR = 128

_R = 2048
_G = NP // _R
_AR = _R // D


def _tc_first_body(x_ref, ws_ref, wd_ref, as_ref, ad_ref, hs_ref, als_ref, ald_ref):
    x = x_ref[...]
    hs = jnp.dot(x, ws_ref[...], preferred_element_type=jnp.float32)
    hs_ref[...] = hs
    als = jnp.dot(hs, as_ref[...], preferred_element_type=jnp.float32)
    als_ref[...] = als.reshape(_AR, D)
    v = jnp.dot(wd_ref[...], ad_ref[...], preferred_element_type=jnp.float32)
    ald_ref[...] = jnp.dot(x, v, preferred_element_type=jnp.float32).reshape(_AR, D)


def _tc_mid_body(acc_ref, s_ref, b_ref, w_ref, as_ref, ad_ref, hs_ref, als_ref, ald_ref):
    num = acc_ref[0] + acc_ref[1]
    s = jnp.sum(s_ref[...], axis=1, keepdims=True)
    h = jnp.maximum(jnp.where(s > 0.0, num / s, 0.0) + b_ref[...], 0.0)
    hs = jnp.dot(h, w_ref[...], preferred_element_type=jnp.float32)
    hs_ref[...] = hs
    als = jnp.dot(hs, as_ref[...], preferred_element_type=jnp.float32)
    als_ref[...] = als.reshape(_AR, D)
    v = jnp.dot(w_ref[...], ad_ref[...], preferred_element_type=jnp.float32)
    ald_ref[...] = jnp.dot(h, v, preferred_element_type=jnp.float32).reshape(_AR, D)


def _tc_last_body(acc_ref, s_ref, b_ref, w1_ref, b1_ref, w2_ref, b2_ref, out_ref):
    num = acc_ref[0] + acc_ref[1]
    s = jnp.sum(s_ref[...], axis=1, keepdims=True)
    h = jnp.maximum(jnp.where(s > 0.0, num / s, 0.0) + b_ref[...], 0.0)
    h = jnp.maximum(jnp.dot(h, w1_ref[...], preferred_element_type=jnp.float32)
                    + b1_ref[...], 0.0)
    out_ref[...] = jnp.dot(h, w2_ref[...], preferred_element_type=jnp.float32) + b2_ref[...]


def _row_blk(i):
    return (i, 0)


def _acc_blk(i):
    return (0, i, 0)


def _full_blk(i):
    return (0, 0)


_W_SPEC = pl.BlockSpec((D, D), _full_blk)
_A_SPEC = pl.BlockSpec((D, 1), _full_blk)
_B_SPEC = pl.BlockSpec((1, D), _full_blk)
_H_SPEC = pl.BlockSpec((_R, D), _row_blk)
_AL_SPEC = pl.BlockSpec((_AR, D), _row_blk)
_ACC_SPEC = pl.BlockSpec((2, _R, D), _acc_blk)
_S_SPEC = pl.BlockSpec((_R, NW), _row_blk)

_PROJ_OUT = (jax.ShapeDtypeStruct((NP, D), jnp.float32),
             jax.ShapeDtypeStruct((NP // D, D), jnp.float32),
             jax.ShapeDtypeStruct((NP // D, D), jnp.float32))

_tc_first = pl.pallas_call(
    _tc_first_body, grid=(_G,),
    in_specs=[_H_SPEC, _W_SPEC, _W_SPEC, _A_SPEC, _A_SPEC],
    out_specs=[_H_SPEC, _AL_SPEC, _AL_SPEC],
    out_shape=_PROJ_OUT)

_tc_mid = pl.pallas_call(
    _tc_mid_body, grid=(_G,),
    in_specs=[_ACC_SPEC, _S_SPEC, _B_SPEC, _W_SPEC, _A_SPEC, _A_SPEC],
    out_specs=[_H_SPEC, _AL_SPEC, _AL_SPEC],
    out_shape=_PROJ_OUT)

_tc_last = pl.pallas_call(
    _tc_last_body, grid=(_G,),
    in_specs=[_ACC_SPEC, _S_SPEC, _B_SPEC, _W_SPEC, _B_SPEC, _W_SPEC, _B_SPEC],
    out_specs=_H_SPEC,
    out_shape=jax.ShapeDtypeStruct((NP, D), jnp.float32))


def _sc_edge_body(hs_hbm, src_hbm, dst_hbm, as_hbm, ad_hbm, out_hbm, s_hbm,
                  src_sb, dst_sb, as_v, ad_v, ee_v, rows_v, s_loc, acc_sp,
                  gsem, ssem):
    cid = lax.axis_index("c")
    sid = lax.axis_index("s")
    wid = sid * 2 + cid

    pltpu.sync_copy(as_hbm.at[pl.ds(0, AL_R)], as_v)
    pltpu.sync_copy(ad_hbm.at[pl.ds(0, AL_R)], ad_v)

    zero16 = jnp.zeros((16,), jnp.float32)

    def _z(r, carry):
        for c in range(D // 16):
            rows_v[0, r, pl.ds(c * 16, 16)] = zero16
        return carry
    lax.fori_loop(0, K, _z, 0)

    def _zs(r, carry):
        for c in range(128 // 16):
            s_loc[r, pl.ds(c * 16, 16)] = zero16
        return carry
    lax.fori_loop(0, AL_R, _zs, 0)

    def _za(t, carry):
        pltpu.sync_copy(rows_v.at[0],
                        acc_sp.at[pl.ds(sid * ROWS_PER_TILE + t * K, K)])
        return carry
    lax.fori_loop(0, ROWS_PER_TILE // K, _za, 0)
    plsc.subcore_barrier()

    def _sblock(sb, carry):
        base = wid * CHUNKS + sb * SB
        pltpu.sync_copy(src_hbm.at[pl.ds(base, SB)], src_sb)
        pltpu.sync_copy(dst_hbm.at[pl.ds(base, SB)], dst_sb)

        pltpu.async_copy(hs_hbm.at[src_sb.at[0]], rows_v.at[0], gsem)

        def _chunk(j, c1):
            b = j % 2
            def _ee(t, c2):
                sv = src_sb[j, pl.ds(t * 16, 16)]
                dv = dst_sb[j, pl.ds(t * 16, 16)]
                e = (plsc.load_gather(as_v, [sv >> 7, sv & 127])
                     + plsc.load_gather(ad_v, [dv >> 7, dv & 127]))
                e = jnp.where(e > 0.0, e, 0.2 * e)
                ee = jnp.exp(e)
                ee_v[t] = ee
                plsc.addupdate_scatter(s_loc, [dv >> 7, dv & 127], ee)
                return c2
            lax.fori_loop(0, K // 16, _ee, 0)

            pltpu.make_async_copy(hs_hbm.at[src_sb.at[j]], rows_v.at[b], gsem).wait()

            @pl.when(j >= 1)
            def _():
                pltpu.make_async_copy(rows_v.at[1 - b],
                                      acc_sp.at[dst_sb.at[j - 1]], ssem).wait()

            @pl.when(j < SB - 1)
            def _():
                pltpu.async_copy(hs_hbm.at[src_sb.at[j + 1]], rows_v.at[1 - b], gsem)

            def _row(t, c2):
                i = t * 2
                w0 = plsc.load_gather(
                    ee_v, [jnp.broadcast_to(i >> 4, (16,)),
                           jnp.broadcast_to(i & 15, (16,))])
                w1 = plsc.load_gather(
                    ee_v, [jnp.broadcast_to((i + 1) >> 4, (16,)),
                           jnp.broadcast_to((i + 1) & 15, (16,))])
                for c in range(D // 16):
                    rows_v[b, i, pl.ds(c * 16, 16)] = (
                        rows_v[b, i, pl.ds(c * 16, 16)] * w0)
                for c in range(D // 16):
                    rows_v[b, i + 1, pl.ds(c * 16, 16)] = (
                        rows_v[b, i + 1, pl.ds(c * 16, 16)] * w1)
                return c2
            lax.fori_loop(0, K // 2, _row, 0)

            pltpu.async_copy(rows_v.at[b], acc_sp.at[dst_sb.at[j]], ssem, add=True)
            return c1
        lax.fori_loop(0, SB, _chunk, 0)

        pltpu.make_async_copy(rows_v.at[(SB - 1) % 2],
                              acc_sp.at[dst_sb.at[SB - 1]], ssem).wait()
        return carry
    lax.fori_loop(0, CHUNKS // SB, _sblock, 0)

    pltpu.sync_copy(s_loc, s_hbm.at[wid])
    plsc.subcore_barrier()

    def _wb(t, carry):
        pltpu.sync_copy(
            acc_sp.at[pl.ds(sid * ROWS_PER_TILE + t * ZR, ZR)],
            out_hbm.at[cid, pl.ds(sid * ROWS_PER_TILE + t * ZR, ZR)])
        return carry
    lax.fori_loop(0, ROWS_PER_TILE // ZR, _wb, 0)


_sc_edge = functools.partial(
    pl.kernel,
    out_type=(jax.ShapeDtypeStruct((2, NP, D), jnp.float32),
              jax.ShapeDtypeStruct((NW, AL_R, 128), jnp.float32)),
    mesh=plsc.VectorSubcoreMesh(core_axis_name="c", subcore_axis_name="s"),
    scratch_types=[
        pltpu.VMEM((SB, K), jnp.int32),
        pltpu.VMEM((SB, K), jnp.int32),
        pltpu.VMEM((AL_R, 128), jnp.float32),
        pltpu.VMEM((AL_R, 128), jnp.float32),
        pltpu.VMEM((K // 16, 16), jnp.float32),
        pltpu.VMEM((2, K, D), jnp.float32),
        pltpu.VMEM((AL_R, 128), jnp.float32),
        pltpu.VMEM_SHARED((NP, D), jnp.float32),
        pltpu.SemaphoreType.DMA,
        pltpu.SemaphoreType.DMA,
    ],
    compiler_params=pltpu.CompilerParams(needs_layout_passes=False),
    )(_sc_edge_body)


def kernel(x, edge_index, W1s, W1d, a1s, a1d, b1, W2, a2s, a2d, b2,
           W3, a3s, a3d, b3, lin1_W, lin1_b, lin2_W, lin2_b):
    f32 = jnp.float32
    x_p = jnp.zeros((NP, D), f32).at[:N_NODES].set(x)
    pad = jnp.full((EP - N_EDGES,), PAD_NODE, jnp.int32)
    src = jnp.concatenate([edge_index[0], pad]).reshape(EP // K, K)
    dst = jnp.concatenate([edge_index[1], pad]).reshape(EP // K, K)

    def col(a):
        return a.reshape(D, 1)

    def row(a, w=D):
        return a.reshape(1, w)

    def tr(sv):
        svt = sv.reshape(NW, AL_R * 128).transpose(1, 0)
        return jnp.zeros((NP, NW), jnp.float32).at[:AL_R * 128].set(svt)

    hs, als, ald = _tc_first(x_p, W1s, W1d, col(a1s), col(a1d))
    acc, sv = _sc_edge(hs, src, dst, als, ald)
    hs, als, ald = _tc_mid(acc, tr(sv), row(b1), W2, col(a2s), col(a2d))
    acc, sv = _sc_edge(hs, src, dst, als, ald)
    hs, als, ald = _tc_mid(acc, tr(sv), row(b2), W3, col(a3s), col(a3d))
    acc, sv = _sc_edge(hs, src, dst, als, ald)

    w2p = jnp.zeros((D, D), f32).at[:, :D_OUT].set(lin2_W)
    b2p = jnp.zeros((D,), f32).at[:D_OUT].set(lin2_b)
    out = _tc_last(acc, tr(sv), row(b3), lin1_W, row(lin1_b), w2p, row(b2p))
    return out[:N_NODES, :D_OUT]
